# baseline (speedup 1.0000x reference)
import functools

import jax
import jax.numpy as jnp
from jax import lax
from jax.experimental import pallas as pl
from jax.experimental.pallas import tpu as pltpu
from jax.experimental.pallas import tpu_sc as plsc

N = 50000
F = 64
H = 32
NNZ = 800000
BATCH = 4096

NS = 16
NC = 2
P = 51200
RPT = P // NS
RC = 128
RCH = RPT // RC
NNZP = 819200
ECH_ROWS = NNZP // 128
TILE_ECH = ECH_ROWS // NS
BLK = 8
TILE_BLKS = TILE_ECH // BLK
NGB = 5

_DNUMS = lax.GatherDimensionNumbers(
    offset_dims=(), collapsed_slice_dims=(0,), start_index_map=(0,))


def _bcast_lane(vec16, lane):
    idx = jnp.full((16, 1), lane, jnp.int32)
    return lax.gather(vec16, idx, _DNUMS, (1,),
                      mode=lax.GatherScatterMode.PROMISE_IN_BOUNDS)


def _propagation_kernel(E, u0p, i0p, D):
    mesh = plsc.VectorSubcoreMesh(core_axis_name="c", subcore_axis_name="s",
                                  num_cores=NC, num_subcores=NS)

    @functools.partial(
        pl.kernel,
        out_type=jax.ShapeDtypeStruct((8, NC, P, H), jnp.float32),
        mesh=mesh,
        compiler_params=pltpu.CompilerParams(use_tc_tiling_on_sc=False,
                                             needs_layout_passes=False),
        scratch_types=[
            pltpu.VMEM_SHARED((P, H), jnp.float32),
            pltpu.VMEM((128, H), jnp.float32),
            pltpu.VMEM((128, H), jnp.float32),
            pltpu.VMEM((128, H), jnp.float32),
            pltpu.VMEM((128, H), jnp.float32),
            pltpu.VMEM((128, H), jnp.float32),
            pltpu.VMEM((BLK, 3, 128), jnp.int32),
            pltpu.SemaphoreType.DMA,
            pltpu.SemaphoreType.DMA,
            pltpu.SemaphoreType.DMA,
        ],
    )
    def body(e_hbm, u0, i0, d_hbm, tab,
             acc, g0, g1, g2, g3, g4, ebuf, gsem, ssem, osem):
        c = lax.axis_index("c")
        s = lax.axis_index("s")
        r0 = s * RPT
        gb = (g0, g1, g2, g3, g4)

        pltpu.sync_copy(u0.at[c].at[pl.ds(r0, RPT)],
                        tab.at[0].at[c].at[pl.ds(r0, RPT)])
        pltpu.sync_copy(i0.at[c].at[pl.ds(r0, RPT)],
                        tab.at[1].at[c].at[pl.ds(r0, RPT)])

        def product_chunk(pb, db):
            @plsc.parallel_loop(0, RC // 8)
            def _(g):
                base = g * 8
                for l in range(8):
                    pb[base + l, pl.ds(0, 16)] = (
                        pb[base + l, pl.ds(0, 16)]
                        * db[base + l, pl.ds(0, 16)])
                    pb[base + l, pl.ds(16, 16)] = (
                        pb[base + l, pl.ds(16, 16)]
                        * db[base + l, pl.ds(16, 16)])

        def step_body(t, _):
            prev = tab.at[t]
            x = tab.at[t ^ 1]
            out = tab.at[t + 2]
            esel = lax.rem(t, 2)
            dsel = esel

            def fire_loads(k, par):
                ld = pltpu.async_copy(
                    prev.at[c].at[pl.ds(r0 + k * RC, RC)], gb[par], gsem)
                dd = pltpu.async_copy(
                    d_hbm.at[dsel].at[pl.ds(r0 + k * RC, RC)],
                    gb[2 + par], gsem)
                return ld, dd

            def handle_chunk(k, par, fire_next):
                pb, db = gb[par], gb[2 + par]
                pltpu.make_async_copy(
                    prev.at[c].at[pl.ds(r0 + k * RC, RC)], pb, gsem).wait()
                pltpu.make_async_copy(
                    d_hbm.at[dsel].at[pl.ds(r0 + k * RC, RC)], db,
                    gsem).wait()
                product_chunk(pb, db)
                st = pltpu.async_copy(
                    pb, acc.at[pl.ds(r0 + k * RC, RC)], osem)
                if fire_next:
                    st.wait()
                    fire_loads(k + 2, par)
                    return None
                return st

            fire_loads(0, 0)
            fire_loads(1, 1)

            def pair_body(p, _):
                handle_chunk(2 * p, 0, True)
                handle_chunk(2 * p + 1, 1, True)
                return _
            lax.fori_loop(0, (RCH - 3) // 2, pair_body, None)
            handle_chunk(RCH - 3, 0, True)
            st_a = handle_chunk(RCH - 2, 1, False)
            st_b = handle_chunk(RCH - 1, 0, False)
            st_a.wait()
            st_b.wait()
            plsc.subcore_barrier()

            xs = x.at[c]

            def blk_body(b, _):
                pltpu.sync_copy(
                    e_hbm.at[esel].at[pl.ds((s * TILE_BLKS + b) * BLK, BLK)],
                    ebuf)

                def fire_gather(j):
                    return pltpu.async_copy(
                        xs.at[ebuf.at[j, 0]], gb[j % NGB], gsem)

                gds = {j: fire_gather(j) for j in range(4)}
                sds = {}
                for j in range(BLK):
                    buf = gb[j % NGB]
                    gds[j].wait()
                    if j >= 1:
                        sds[j - 1].wait()
                    if j + 4 < BLK:
                        gds[j + 4] = fire_gather(j + 4)

                    @plsc.parallel_loop(0, 8)
                    def _(g):
                        vals16 = plsc.bitcast(
                            ebuf[j, 2, pl.ds(g * 16, 16)], jnp.float32)
                        base = g * 16
                        for l in range(16):
                            vv = _bcast_lane(vals16, l)
                            buf[base + l, pl.ds(0, 16)] = (
                                buf[base + l, pl.ds(0, 16)] * vv)
                            buf[base + l, pl.ds(16, 16)] = (
                                buf[base + l, pl.ds(16, 16)] * vv)

                    sds[j] = pltpu.async_copy(
                        buf, acc.at[ebuf.at[j, 1]], ssem, add=True)
                sds[BLK - 1].wait()
                return _
            lax.fori_loop(0, TILE_BLKS, blk_body, None)
            plsc.subcore_barrier()

            pltpu.sync_copy(acc.at[pl.ds(r0, RPT)],
                            out.at[c].at[pl.ds(r0, RPT)])
            plsc.subcore_barrier()
            return _

        plsc.subcore_barrier()
        lax.fori_loop(0, 6, step_body, None)

    return body(E, u0p, i0p, D)


def _bpr_gather_kernel(user2, itemi2, itemj2, tab):
    mesh = plsc.VectorSubcoreMesh(core_axis_name="c", subcore_axis_name="s",
                                  num_cores=NC, num_subcores=NS)
    out = jax.ShapeDtypeStruct((BATCH, 4 * F), jnp.float32)

    @functools.partial(
        pl.kernel,
        out_type=[out] * 3,
        mesh=mesh,
        compiler_params=pltpu.CompilerParams(use_tc_tiling_on_sc=False,
                                             needs_layout_passes=False),
        scratch_types=[
            pltpu.VMEM((128,), jnp.int32),
            pltpu.VMEM((128, H), jnp.float32),
            pltpu.SemaphoreType.DMA,
        ],
    )
    def body(uu, ti, tj, tabs, ue_o, ii_o, ij_o, idxv, buf, sem):
        c = lax.axis_index("c")
        s = lax.axis_index("s")
        wid = s * NC + c
        r0 = wid * 128
        for idx_hbm, base_t, o in ((uu, 0, ue_o), (ti, 1, ii_o),
                                   (tj, 1, ij_o)):
            pltpu.sync_copy(idx_hbm.at[wid], idxv)
            for t in range(4):
                for h in range(2):
                    pltpu.async_copy(
                        tabs.at[base_t + 2 * t].at[h].at[idxv], buf,
                        sem).wait()
                    pltpu.sync_copy(
                        buf,
                        o.at[pl.ds(r0, 128), pl.ds(t * F + h * H, H)])

    return body(user2, itemi2, itemj2, tab)


def _loss_body(ue_ref, ii_ref, ij_ref, pi_ref, pj_ref, l_ref, l2_ref):
    ue = ue_ref[...]
    ii = ii_ref[...]
    ij = ij_ref[...]
    pi = jnp.sum(ue * ii, axis=1, keepdims=True)
    pj = jnp.sum(ue * ij, axis=1, keepdims=True)
    l2 = 0.01 * jnp.sum(ue * ue + ii * ii + ij * ij, axis=1, keepdims=True)
    pi_ref[...] = pi
    pj_ref[...] = pj
    d = pi - pj
    loss2 = jnp.mean(jnp.log(1.0 + jnp.exp(-d)))
    l2_ref[...] = jnp.broadcast_to(loss2, (1, 1))
    l_ref[...] = jnp.broadcast_to(loss2 + jnp.mean(l2), (1, 1))


def _pack_table(x):
    xp = jnp.pad(x, ((0, P - N), (0, 0)))
    return jnp.transpose(xp.reshape(P, 2, H), (1, 0, 2))


def kernel(user, item_i, item_j, embed_user, embed_item, d_i, d_j,
           edge_u, edge_i, edge_vals):
    u0p = _pack_table(embed_user)
    i0p = _pack_table(embed_item)
    dip = jnp.pad(d_i[:, :H], ((0, P - N), (0, 0)))
    djp = jnp.pad(d_j[:, :H], ((0, P - N), (0, 0)))
    D = jnp.stack([dip, djp])
    eu2 = jnp.pad(edge_u.astype(jnp.int32), (0, NNZP - NNZ),
                  constant_values=P - 1).reshape(ECH_ROWS, 128)
    ei2 = jnp.pad(edge_i.astype(jnp.int32), (0, NNZP - NNZ),
                  constant_values=P - 1).reshape(ECH_ROWS, 128)
    vb2 = lax.bitcast_convert_type(
        jnp.pad(edge_vals, (0, NNZP - NNZ)), jnp.int32).reshape(ECH_ROWS, 128)
    E = jnp.stack([jnp.stack([ei2, eu2, vb2], axis=1),
                   jnp.stack([eu2, ei2, vb2], axis=1)])

    tab = _propagation_kernel(E, u0p, i0p, D)

    user2 = user.astype(jnp.int32).reshape(32, 128)
    itemi2 = item_i.astype(jnp.int32).reshape(32, 128)
    itemj2 = item_j.astype(jnp.int32).reshape(32, 128)
    ue, ii, ij = _bpr_gather_kernel(user2, itemi2, itemj2, tab)

    pi, pj, loss, loss2 = pl.pallas_call(
        _loss_body,
        out_shape=[
            jax.ShapeDtypeStruct((BATCH, 1), jnp.float32),
            jax.ShapeDtypeStruct((BATCH, 1), jnp.float32),
            jax.ShapeDtypeStruct((1, 1), jnp.float32),
            jax.ShapeDtypeStruct((1, 1), jnp.float32),
        ],
    )(ue, ii, ij)

    return (pi.reshape(BATCH), pj.reshape(BATCH),
            loss.reshape(()), loss2.reshape(()))


# speedup vs baseline: 1.3642x; 1.3642x over previous
import functools

import jax
import jax.numpy as jnp
from jax import lax
from jax.experimental import pallas as pl
from jax.experimental.pallas import tpu as pltpu
from jax.experimental.pallas import tpu_sc as plsc

N = 50000
F = 64
H = 32
NNZ = 800000
BATCH = 4096

NS = 16
NC = 2
P = 51200
RPT = P // NS
RC = 128
RCH = RPT // RC
NNZP = 819200
ECH_ROWS = NNZP // 128
TILE_ECH = ECH_ROWS // NS
BLK = 8
TILE_BLKS = TILE_ECH // BLK
NGB = 5

_DNUMS = lax.GatherDimensionNumbers(
    offset_dims=(), collapsed_slice_dims=(0,), start_index_map=(0,))
_PK = plsc.PackFormat.INTERLEAVED


def _bcast_lane(vec16, lane):
    idx = jnp.full((16, 1), lane, jnp.int32)
    return lax.gather(vec16, idx, _DNUMS, (1,),
                      mode=lax.GatherScatterMode.PROMISE_IN_BOUNDS)


def _propagation_kernel(E, u0p, i0p, D):
    mesh = plsc.VectorSubcoreMesh(core_axis_name="c", subcore_axis_name="s",
                                  num_cores=NC, num_subcores=NS)

    @functools.partial(
        pl.kernel,
        out_type=jax.ShapeDtypeStruct((8, NC, P, H), jnp.bfloat16),
        mesh=mesh,
        compiler_params=pltpu.CompilerParams(use_tc_tiling_on_sc=False,
                                             needs_layout_passes=False),
        scratch_types=[
            pltpu.VMEM_SHARED((P, H), jnp.float32),
            pltpu.VMEM((RC, H), jnp.bfloat16),
            pltpu.VMEM((RC, H), jnp.bfloat16),
            pltpu.VMEM((RC, H), jnp.bfloat16),
            pltpu.VMEM((RC, H), jnp.bfloat16),
            pltpu.VMEM((RC, H), jnp.bfloat16),
            pltpu.VMEM((RC, H), jnp.float32),
            pltpu.VMEM((RC, H), jnp.float32),
            pltpu.VMEM((BLK, 3, 128), jnp.int32),
            pltpu.SemaphoreType.DMA,
            pltpu.SemaphoreType.DMA,
            pltpu.SemaphoreType.DMA,
        ],
    )
    def body(e_hbm, u0, i0, d_hbm, tab,
             acc, g0, g1, g2, g3, g4, sb0, sb1, ebuf, gsem, ssem, osem):
        c = lax.axis_index("c")
        s = lax.axis_index("s")
        r0 = s * RPT
        gb = (g0, g1, g2, g3, g4)
        sb = (sb0, sb1)

        pltpu.sync_copy(u0.at[c].at[pl.ds(r0, RPT)],
                        tab.at[0].at[c].at[pl.ds(r0, RPT)])
        pltpu.sync_copy(i0.at[c].at[pl.ds(r0, RPT)],
                        tab.at[1].at[c].at[pl.ds(r0, RPT)])

        def step_body(t, _):
            prev = tab.at[t]
            x = tab.at[t ^ 1]
            out = tab.at[t + 2]
            esel = lax.rem(t, 2)
            dsel = esel

            def fire_loads(k, par):
                ld = pltpu.async_copy(
                    prev.at[c].at[pl.ds(r0 + k * RC, RC)], gb[par], gsem)
                dd = pltpu.async_copy(
                    d_hbm.at[dsel].at[pl.ds(r0 + k * RC, RC)],
                    gb[2 + par], gsem)
                return ld, dd

            def handle_chunk(k, par, fire_next):
                pb, db, ob = gb[par], gb[2 + par], sb[par]
                pltpu.make_async_copy(
                    prev.at[c].at[pl.ds(r0 + k * RC, RC)], pb, gsem).wait()
                pltpu.make_async_copy(
                    d_hbm.at[dsel].at[pl.ds(r0 + k * RC, RC)], db,
                    gsem).wait()

                @plsc.parallel_loop(0, RC // 8)
                def _(g):
                    base = g * 8
                    for l in range(8):
                        pa, pbh = plsc.unpack(pb[base + l, :], format=_PK)
                        da, dbh = plsc.unpack(db[base + l, :], format=_PK)
                        ob[base + l, pl.ds(0, 16)] = pa * da
                        ob[base + l, pl.ds(16, 16)] = pbh * dbh

                st = pltpu.async_copy(
                    ob, acc.at[pl.ds(r0 + k * RC, RC)], osem)
                if fire_next:
                    st.wait()
                    fire_loads(k + 2, par)
                    return None
                return st

            fire_loads(0, 0)
            fire_loads(1, 1)

            def pair_body(p, _):
                handle_chunk(2 * p, 0, True)
                handle_chunk(2 * p + 1, 1, True)
                return _
            lax.fori_loop(0, (RCH - 3) // 2, pair_body, None)
            handle_chunk(RCH - 3, 0, True)
            st_a = handle_chunk(RCH - 2, 1, False)
            st_b = handle_chunk(RCH - 1, 0, False)
            st_a.wait()
            st_b.wait()
            plsc.subcore_barrier()

            xs = x.at[c]

            def blk_body(b, _):
                pltpu.sync_copy(
                    e_hbm.at[esel].at[pl.ds((s * TILE_BLKS + b) * BLK, BLK)],
                    ebuf)

                def fire_gather(j):
                    return pltpu.async_copy(
                        xs.at[ebuf.at[j, 0]], gb[j % NGB], gsem)

                gds = {j: fire_gather(j) for j in range(3)}
                sds = {}
                for j in range(BLK):
                    buf = gb[j % NGB]
                    ob = sb[j % 2]
                    gds[j].wait()
                    if j >= 2:
                        sds[j - 2].wait()

                    @plsc.parallel_loop(0, 8)
                    def _(g):
                        vals16 = plsc.bitcast(
                            ebuf[j, 2, pl.ds(g * 16, 16)], jnp.float32)
                        base = g * 16
                        for l in range(16):
                            vv = _bcast_lane(vals16, l)
                            lo, hi = plsc.unpack(buf[base + l, :],
                                                 format=_PK)
                            ob[base + l, pl.ds(0, 16)] = lo * vv
                            ob[base + l, pl.ds(16, 16)] = hi * vv

                    if j + 3 < BLK:
                        gds[j + 3] = fire_gather(j + 3)
                    sds[j] = pltpu.async_copy(
                        ob, acc.at[ebuf.at[j, 1]], ssem, add=True)
                sds[BLK - 2].wait()
                sds[BLK - 1].wait()
                return _
            lax.fori_loop(0, TILE_BLKS, blk_body, None)
            plsc.subcore_barrier()

            def wb(k, _):
                rr = r0 + k * RC
                pltpu.sync_copy(acc.at[pl.ds(rr, RC)], sb0)

                @plsc.parallel_loop(0, RC // 8)
                def _(g):
                    base = g * 8
                    for l in range(8):
                        g0[base + l, :] = plsc.pack(
                            sb0[base + l, pl.ds(0, 16)],
                            sb0[base + l, pl.ds(16, 16)], format=_PK)

                pltpu.sync_copy(g0, out.at[c].at[pl.ds(rr, RC)])
                return _
            lax.fori_loop(0, RCH, wb, None)
            plsc.subcore_barrier()
            return _

        plsc.subcore_barrier()
        lax.fori_loop(0, 6, step_body, None)

    return body(E, u0p, i0p, D)


def _bpr_gather_kernel(user2, itemi2, itemj2, tab):
    mesh = plsc.VectorSubcoreMesh(core_axis_name="c", subcore_axis_name="s",
                                  num_cores=NC, num_subcores=NS)
    out = jax.ShapeDtypeStruct((BATCH, 4 * F), jnp.bfloat16)

    @functools.partial(
        pl.kernel,
        out_type=[out] * 3,
        mesh=mesh,
        compiler_params=pltpu.CompilerParams(use_tc_tiling_on_sc=False,
                                             needs_layout_passes=False),
        scratch_types=[
            pltpu.VMEM((128,), jnp.int32),
            pltpu.VMEM((128, H), jnp.bfloat16),
            pltpu.SemaphoreType.DMA,
        ],
    )
    def body(uu, ti, tj, tabs, ue_o, ii_o, ij_o, idxv, buf, sem):
        c = lax.axis_index("c")
        s = lax.axis_index("s")
        wid = s * NC + c
        r0 = wid * 128
        for idx_hbm, base_t, o in ((uu, 0, ue_o), (ti, 1, ii_o),
                                   (tj, 1, ij_o)):
            pltpu.sync_copy(idx_hbm.at[wid], idxv)
            for t in range(4):
                for h in range(2):
                    pltpu.async_copy(
                        tabs.at[base_t + 2 * t].at[h].at[idxv], buf,
                        sem).wait()
                    pltpu.sync_copy(
                        buf, o.at[pl.ds(r0, 128), pl.ds(t * F + h * H, H)])

    return body(user2, itemi2, itemj2, tab)


def _loss_body(ue_ref, ii_ref, ij_ref, pi_ref, pj_ref, l_ref, l2_ref):
    ue = ue_ref[...].astype(jnp.float32)
    ii = ii_ref[...].astype(jnp.float32)
    ij = ij_ref[...].astype(jnp.float32)
    pi = jnp.sum(ue * ii, axis=1, keepdims=True)
    pj = jnp.sum(ue * ij, axis=1, keepdims=True)
    l2 = 0.01 * jnp.sum(ue * ue + ii * ii + ij * ij, axis=1, keepdims=True)
    pi_ref[...] = pi
    pj_ref[...] = pj
    d = pi - pj
    loss2 = jnp.mean(jnp.log(1.0 + jnp.exp(-d)))
    l2_ref[...] = jnp.broadcast_to(loss2, (1, 1))
    l_ref[...] = jnp.broadcast_to(loss2 + jnp.mean(l2), (1, 1))


def _pack_table(x):
    xp = jnp.pad(x, ((0, P - N), (0, 0))).astype(jnp.bfloat16)
    return jnp.transpose(xp.reshape(P, 2, H), (1, 0, 2))


def kernel(user, item_i, item_j, embed_user, embed_item, d_i, d_j,
           edge_u, edge_i, edge_vals):
    u0p = _pack_table(embed_user)
    i0p = _pack_table(embed_item)
    dip = jnp.pad(d_i[:, :H], ((0, P - N), (0, 0)))
    djp = jnp.pad(d_j[:, :H], ((0, P - N), (0, 0)))
    D = jnp.stack([dip, djp]).astype(jnp.bfloat16)
    eu2 = jnp.pad(edge_u.astype(jnp.int32), (0, NNZP - NNZ),
                  constant_values=P - 1).reshape(ECH_ROWS, 128)
    ei2 = jnp.pad(edge_i.astype(jnp.int32), (0, NNZP - NNZ),
                  constant_values=P - 1).reshape(ECH_ROWS, 128)
    vb2 = lax.bitcast_convert_type(
        jnp.pad(edge_vals, (0, NNZP - NNZ)), jnp.int32).reshape(ECH_ROWS, 128)
    E = jnp.stack([jnp.stack([ei2, eu2, vb2], axis=1),
                   jnp.stack([eu2, ei2, vb2], axis=1)])

    tab = _propagation_kernel(E, u0p, i0p, D)

    user2 = user.astype(jnp.int32).reshape(32, 128)
    itemi2 = item_i.astype(jnp.int32).reshape(32, 128)
    itemj2 = item_j.astype(jnp.int32).reshape(32, 128)
    ue, ii, ij = _bpr_gather_kernel(user2, itemi2, itemj2, tab)

    pi, pj, loss, loss2 = pl.pallas_call(
        _loss_body,
        out_shape=[
            jax.ShapeDtypeStruct((BATCH, 1), jnp.float32),
            jax.ShapeDtypeStruct((BATCH, 1), jnp.float32),
            jax.ShapeDtypeStruct((1, 1), jnp.float32),
            jax.ShapeDtypeStruct((1, 1), jnp.float32),
        ],
    )(ue, ii, ij)

    return (pi.reshape(BATCH), pj.reshape(BATCH),
            loss.reshape(()), loss2.reshape(()))


# speedup vs baseline: 1.3995x; 1.0259x over previous
import functools

import jax
import jax.numpy as jnp
from jax import lax
from jax.experimental import pallas as pl
from jax.experimental.pallas import tpu as pltpu
from jax.experimental.pallas import tpu_sc as plsc

N = 50000
F = 64
H = 32
NNZ = 800000
BATCH = 4096

NS = 16
NC = 2
P = 51200
RPT = P // NS
RC = 128
RCH = RPT // RC
NNZP = 819200
ECH_ROWS = NNZP // 128
TILE_ECH = ECH_ROWS // NS
BLK = 8
TILE_BLKS = TILE_ECH // BLK
NGB = 6

_DNUMS = lax.GatherDimensionNumbers(
    offset_dims=(), collapsed_slice_dims=(0,), start_index_map=(0,))
_PK = plsc.PackFormat.INTERLEAVED


def _bcast_lane(vec16, lane):
    idx = jnp.full((16, 1), lane, jnp.int32)
    return lax.gather(vec16, idx, _DNUMS, (1,),
                      mode=lax.GatherScatterMode.PROMISE_IN_BOUNDS)


def _propagation_kernel(E, u0p, i0p, D):
    mesh = plsc.VectorSubcoreMesh(core_axis_name="c", subcore_axis_name="s",
                                  num_cores=NC, num_subcores=NS)

    @functools.partial(
        pl.kernel,
        out_type=jax.ShapeDtypeStruct((8, NC, P, H), jnp.bfloat16),
        mesh=mesh,
        compiler_params=pltpu.CompilerParams(use_tc_tiling_on_sc=False,
                                             needs_layout_passes=False),
        scratch_types=[
            pltpu.VMEM_SHARED((P, H), jnp.float32),
            pltpu.VMEM((RC, H), jnp.bfloat16),
            pltpu.VMEM((RC, H), jnp.bfloat16),
            pltpu.VMEM((RC, H), jnp.bfloat16),
            pltpu.VMEM((RC, H), jnp.bfloat16),
            pltpu.VMEM((RC, H), jnp.bfloat16),
            pltpu.VMEM((RC, H), jnp.bfloat16),
            pltpu.VMEM((RC, H), jnp.float32),
            pltpu.VMEM((RC, H), jnp.float32),
            pltpu.VMEM((BLK, 3, 128), jnp.int32),
            pltpu.SemaphoreType.DMA,
            pltpu.SemaphoreType.DMA,
            pltpu.SemaphoreType.DMA,
        ],
    )
    def body(e_hbm, u0, i0, d_hbm, tab,
             acc, g0, g1, g2, g3, g4, g5, sb0, sb1, ebuf, gsem, ssem, osem):
        c = lax.axis_index("c")
        s = lax.axis_index("s")
        r0 = s * RPT
        gb = (g0, g1, g2, g3, g4, g5)
        sb = (sb0, sb1)

        pltpu.sync_copy(u0.at[c].at[pl.ds(r0, RPT)],
                        tab.at[0].at[c].at[pl.ds(r0, RPT)])
        pltpu.sync_copy(i0.at[c].at[pl.ds(r0, RPT)],
                        tab.at[1].at[c].at[pl.ds(r0, RPT)])

        def step_body(t, _):
            prev = tab.at[t]
            x = tab.at[t ^ 1]
            out = tab.at[t + 2]
            esel = lax.rem(t, 2)
            dsel = esel

            def fire_loads(k, par):
                ld = pltpu.async_copy(
                    prev.at[c].at[pl.ds(r0 + k * RC, RC)], gb[par], gsem)
                dd = pltpu.async_copy(
                    d_hbm.at[dsel].at[pl.ds(r0 + k * RC, RC)],
                    gb[2 + par], gsem)
                return ld, dd

            def handle_chunk(k, par, fire_next):
                pb, db, ob = gb[par], gb[2 + par], sb[par]
                pltpu.make_async_copy(
                    prev.at[c].at[pl.ds(r0 + k * RC, RC)], pb, gsem).wait()
                pltpu.make_async_copy(
                    d_hbm.at[dsel].at[pl.ds(r0 + k * RC, RC)], db,
                    gsem).wait()

                @plsc.parallel_loop(0, RC // 8)
                def _(g):
                    base = g * 8
                    for l in range(8):
                        pa, pbh = plsc.unpack(pb[base + l, :], format=_PK)
                        da, dbh = plsc.unpack(db[base + l, :], format=_PK)
                        ob[base + l, pl.ds(0, 16)] = pa * da
                        ob[base + l, pl.ds(16, 16)] = pbh * dbh

                st = pltpu.async_copy(
                    ob, acc.at[pl.ds(r0 + k * RC, RC)], osem)
                if fire_next:
                    st.wait()
                    fire_loads(k + 2, par)
                    return None
                return st

            fire_loads(0, 0)
            fire_loads(1, 1)

            def pair_body(p, _):
                handle_chunk(2 * p, 0, True)
                handle_chunk(2 * p + 1, 1, True)
                return _
            lax.fori_loop(0, (RCH - 3) // 2, pair_body, None)
            handle_chunk(RCH - 3, 0, True)
            st_a = handle_chunk(RCH - 2, 1, False)
            st_b = handle_chunk(RCH - 1, 0, False)
            st_a.wait()
            st_b.wait()
            plsc.subcore_barrier()

            xs = x.at[c]

            def blk_body(b, _):
                pltpu.sync_copy(
                    e_hbm.at[esel].at[pl.ds((s * TILE_BLKS + b) * BLK, BLK)],
                    ebuf)

                def fire_gather(j):
                    return pltpu.async_copy(
                        xs.at[ebuf.at[j, 0]], gb[j % NGB], gsem)

                gds = {j: fire_gather(j) for j in range(4)}
                sds = {}
                for j in range(BLK):
                    buf = gb[j % NGB]
                    ob = sb[j % 2]
                    gds[j].wait()
                    if j >= 2:
                        sds[j - 2].wait()

                    @plsc.parallel_loop(0, 8)
                    def _(g):
                        vals16 = plsc.bitcast(
                            ebuf[j, 2, pl.ds(g * 16, 16)], jnp.float32)
                        base = g * 16
                        for l in range(16):
                            vv = _bcast_lane(vals16, l)
                            lo, hi = plsc.unpack(buf[base + l, :],
                                                 format=_PK)
                            ob[base + l, pl.ds(0, 16)] = lo * vv
                            ob[base + l, pl.ds(16, 16)] = hi * vv

                    if j + 4 < BLK:
                        gds[j + 4] = fire_gather(j + 4)
                    sds[j] = pltpu.async_copy(
                        ob, acc.at[ebuf.at[j, 1]], ssem, add=True)
                sds[BLK - 2].wait()
                sds[BLK - 1].wait()
                return _
            lax.fori_loop(0, TILE_BLKS, blk_body, None)
            plsc.subcore_barrier()

            def wb(k, _):
                rr = r0 + k * RC
                pltpu.sync_copy(acc.at[pl.ds(rr, RC)], sb0)

                @plsc.parallel_loop(0, RC // 8)
                def _(g):
                    base = g * 8
                    for l in range(8):
                        g0[base + l, :] = plsc.pack(
                            sb0[base + l, pl.ds(0, 16)],
                            sb0[base + l, pl.ds(16, 16)], format=_PK)

                pltpu.sync_copy(g0, out.at[c].at[pl.ds(rr, RC)])
                return _
            lax.fori_loop(0, RCH, wb, None)
            plsc.subcore_barrier()
            return _

        plsc.subcore_barrier()
        lax.fori_loop(0, 6, step_body, None)

    return body(E, u0p, i0p, D)


def _bpr_gather_kernel(user2, itemi2, itemj2, tab):
    mesh = plsc.VectorSubcoreMesh(core_axis_name="c", subcore_axis_name="s",
                                  num_cores=NC, num_subcores=NS)
    out = jax.ShapeDtypeStruct((BATCH, 4 * F), jnp.bfloat16)

    @functools.partial(
        pl.kernel,
        out_type=[out] * 3,
        mesh=mesh,
        compiler_params=pltpu.CompilerParams(use_tc_tiling_on_sc=False,
                                             needs_layout_passes=False),
        scratch_types=[
            pltpu.VMEM((128,), jnp.int32),
            pltpu.VMEM((128, H), jnp.bfloat16),
            pltpu.SemaphoreType.DMA,
        ],
    )
    def body(uu, ti, tj, tabs, ue_o, ii_o, ij_o, idxv, buf, sem):
        c = lax.axis_index("c")
        s = lax.axis_index("s")
        wid = s * NC + c
        r0 = wid * 128
        for idx_hbm, base_t, o in ((uu, 0, ue_o), (ti, 1, ii_o),
                                   (tj, 1, ij_o)):
            pltpu.sync_copy(idx_hbm.at[wid], idxv)
            for t in range(4):
                for h in range(2):
                    pltpu.async_copy(
                        tabs.at[base_t + 2 * t].at[h].at[idxv], buf,
                        sem).wait()
                    pltpu.sync_copy(
                        buf, o.at[pl.ds(r0, 128), pl.ds(t * F + h * H, H)])

    return body(user2, itemi2, itemj2, tab)


def _loss_body(ue_ref, ii_ref, ij_ref, pi_ref, pj_ref, l_ref, l2_ref):
    ue = ue_ref[...].astype(jnp.float32)
    ii = ii_ref[...].astype(jnp.float32)
    ij = ij_ref[...].astype(jnp.float32)
    pi = jnp.sum(ue * ii, axis=1, keepdims=True)
    pj = jnp.sum(ue * ij, axis=1, keepdims=True)
    l2 = 0.01 * jnp.sum(ue * ue + ii * ii + ij * ij, axis=1, keepdims=True)
    pi_ref[...] = pi
    pj_ref[...] = pj
    d = pi - pj
    loss2 = jnp.mean(jnp.log(1.0 + jnp.exp(-d)))
    l2_ref[...] = jnp.broadcast_to(loss2, (1, 1))
    l_ref[...] = jnp.broadcast_to(loss2 + jnp.mean(l2), (1, 1))


def _pack_table(x):
    xp = jnp.pad(x, ((0, P - N), (0, 0))).astype(jnp.bfloat16)
    return jnp.transpose(xp.reshape(P, 2, H), (1, 0, 2))


def kernel(user, item_i, item_j, embed_user, embed_item, d_i, d_j,
           edge_u, edge_i, edge_vals):
    u0p = _pack_table(embed_user)
    i0p = _pack_table(embed_item)
    dip = jnp.pad(d_i[:, :H], ((0, P - N), (0, 0)))
    djp = jnp.pad(d_j[:, :H], ((0, P - N), (0, 0)))
    D = jnp.stack([dip, djp]).astype(jnp.bfloat16)
    eu2 = jnp.pad(edge_u.astype(jnp.int32), (0, NNZP - NNZ),
                  constant_values=P - 1).reshape(ECH_ROWS, 128)
    ei2 = jnp.pad(edge_i.astype(jnp.int32), (0, NNZP - NNZ),
                  constant_values=P - 1).reshape(ECH_ROWS, 128)
    vb2 = lax.bitcast_convert_type(
        jnp.pad(edge_vals, (0, NNZP - NNZ)), jnp.int32).reshape(ECH_ROWS, 128)
    E = jnp.stack([jnp.stack([ei2, eu2, vb2], axis=1),
                   jnp.stack([eu2, ei2, vb2], axis=1)])

    tab = _propagation_kernel(E, u0p, i0p, D)

    user2 = user.astype(jnp.int32).reshape(32, 128)
    itemi2 = item_i.astype(jnp.int32).reshape(32, 128)
    itemj2 = item_j.astype(jnp.int32).reshape(32, 128)
    ue, ii, ij = _bpr_gather_kernel(user2, itemi2, itemj2, tab)

    pi, pj, loss, loss2 = pl.pallas_call(
        _loss_body,
        out_shape=[
            jax.ShapeDtypeStruct((BATCH, 1), jnp.float32),
            jax.ShapeDtypeStruct((BATCH, 1), jnp.float32),
            jax.ShapeDtypeStruct((1, 1), jnp.float32),
            jax.ShapeDtypeStruct((1, 1), jnp.float32),
        ],
    )(ue, ii, ij)

    return (pi.reshape(BATCH), pj.reshape(BATCH),
            loss.reshape(()), loss2.reshape(()))
